# Initial kernel scaffold; baseline (speedup 1.0000x reference)
import jax
import jax.numpy as jnp
from jax.experimental import pallas as pl


def kernel(userIdx, servIdx, user_as_map, user_re_map, serv_as_map, serv_re_map, serv_pr_map, user_emb, uas_emb, ure_emb, serv_emb, sas_emb, sre_emb, spr_emb, user_ln_w, user_ln_b, serv_ln_w, serv_ln_b, norm_w, norm_b):
    raise NotImplementedError("write your pallas kernel here")


# speedup vs baseline: 1.6643x; 1.6643x over previous
import functools

import jax
import jax.numpy as jnp
from jax import lax
from jax.experimental import pallas as pl
from jax.experimental.pallas import tpu as pltpu
from jax.experimental.pallas import tpu_sc as plsc

R = 128
B = 16384
NC = 2
NS = 16
NW = NC * NS
L = 16
SPW = B // NW
C = 64
NCHUNK = SPW // C
NG = C // L
EPS = 1e-5


def _rsqrt(x):
    i = plsc.bitcast(x, jnp.int32)
    i = jnp.int32(0x5F3759DF) - (i >> 1)
    y = plsc.bitcast(i, jnp.float32)
    for _ in range(3):
        y = y * (1.5 - 0.5 * x * y * y)
    return y


def _sc_body(uidx_hbm, sidx_hbm, umapA_hbm, umapB_hbm,
             smapA_hbm, smapB_hbm, smapC_hbm,
             uemb_hbm, uas_hbm, ure_hbm,
             semb_hbm, sas_hbm, sre_hbm, spr_hbm,
             prep_hbm,
             out_hbm,
             uidx_v, sidx_v, uasi_v, urei_v, sasi_v, srei_v, spri_v,
             b_u, b_uas, b_ure, b_s, b_sas, b_sre, b_spr,
             prep_v, out_v, sem):
    wid = lax.axis_index("s") * NC + lax.axis_index("c")
    base = wid * SPW

    pltpu.async_copy(prep_hbm, prep_v, sem).wait()

    def _sum_param(k, acc_init):
        def body(f, acc):
            return acc + prep_v[pl.ds(k * (R * L) + f * L, L)]
        return lax.fori_loop(0, R, body, acc_init)
    Sw = _sum_param(4, jnp.zeros((L,), jnp.float32))
    Sb = _sum_param(5, jnp.zeros((L,), jnp.float32))

    for c in range(NCHUNK):
        off = base + c * C
        pltpu.async_copy(uidx_hbm.at[pl.ds(off, C)], uidx_v, sem).wait()
        pltpu.async_copy(sidx_hbm.at[pl.ds(off, C)], sidx_v, sem).wait()
        pltpu.async_copy(umapA_hbm.at[uidx_v], uasi_v, sem).wait()
        pltpu.async_copy(umapB_hbm.at[uidx_v], urei_v, sem).wait()
        pltpu.async_copy(smapA_hbm.at[sidx_v], sasi_v, sem).wait()
        pltpu.async_copy(smapB_hbm.at[sidx_v], srei_v, sem).wait()
        pltpu.async_copy(smapC_hbm.at[sidx_v], spri_v, sem).wait()
        pltpu.async_copy(uemb_hbm.at[uidx_v], b_u, sem).wait()
        pltpu.async_copy(uas_hbm.at[uasi_v], b_uas, sem).wait()
        pltpu.async_copy(ure_hbm.at[urei_v], b_ure, sem).wait()
        pltpu.async_copy(semb_hbm.at[sidx_v], b_s, sem).wait()
        pltpu.async_copy(sas_hbm.at[sasi_v], b_sas, sem).wait()
        pltpu.async_copy(sre_hbm.at[srei_v], b_sre, sem).wait()
        pltpu.async_copy(spr_hbm.at[spri_v], b_spr, sem).wait()

        for g in range(NG):
            rows = lax.iota(jnp.int32, L) + jnp.int32(g * L)

            def load_us(f):
                col = jnp.full((L,), f, jnp.int32)
                u = (plsc.load_gather(b_u, [rows, col])
                     + plsc.load_gather(b_uas, [rows, col])
                     + plsc.load_gather(b_ure, [rows, col]))
                s = (plsc.load_gather(b_s, [rows, col])
                     + plsc.load_gather(b_sas, [rows, col])
                     + plsc.load_gather(b_sre, [rows, col])
                     + plsc.load_gather(b_spr, [rows, col]))
                return u, s

            def p1(f, acc):
                su, suu, ss, sss = acc
                u, s = load_us(f)
                return su + u, suu + u * u, ss + s, sss + s * s
            z = jnp.zeros((L,), jnp.float32)
            su, suu, ss, sss = lax.fori_loop(0, R, p1, (z, z, z, z))
            inv_r = jnp.float32(1.0 / R)
            mu = su * inv_r
            ms = ss * inv_r
            iu = _rsqrt(suu * inv_r - mu * mu + EPS)
            isv = _rsqrt(sss * inv_r - ms * ms + EPS)

            def p2(f, acc):
                P, Q, W = acc
                u, s = load_us(f)
                poff = f * L
                uw = prep_v[pl.ds(poff, L)]
                ub = prep_v[pl.ds((R * L) + poff, L)]
                sw = prep_v[pl.ds(2 * (R * L) + poff, L)]
                sb = prep_v[pl.ds(3 * (R * L) + poff, L)]
                w = prep_v[pl.ds(4 * (R * L) + poff, L)]
                un = (u - mu) * iu * uw + ub
                sn = (s - ms) * isv * sw + sb
                prod = un * sn
                return P + prod, Q + prod * prod, W + prod * w
            P, Q, W = lax.fori_loop(0, R, p2, (z, z, z))

            m3 = P * inv_r
            i3 = _rsqrt(Q * inv_r - m3 * m3 + EPS)
            tmp = i3 * (W - m3 * Sw) + Sb
            pred = 1.0 / (1.0 + jnp.exp(-tmp))
            out_v[pl.ds(c * C + g * L, L)] = pred

    pltpu.async_copy(out_v, out_hbm.at[pl.ds(base, SPW)], sem).wait()


@jax.jit
def _csmf_sc(uidx, sidx, umapA, umapB, smapA, smapB, smapC,
             uemb, uas, ure, semb, sas, sre, spr, prep_flat):
    mesh = plsc.VectorSubcoreMesh(core_axis_name="c", subcore_axis_name="s",
                                  num_cores=NC, num_subcores=NS)
    f = pl.kernel(
        _sc_body,
        out_type=jax.ShapeDtypeStruct((B,), jnp.float32),
        mesh=mesh,
        compiler_params=pltpu.CompilerParams(needs_layout_passes=False),
        scratch_types=[
            pltpu.VMEM((C,), jnp.int32),
            pltpu.VMEM((C,), jnp.int32),
            pltpu.VMEM((C,), jnp.int32),
            pltpu.VMEM((C,), jnp.int32),
            pltpu.VMEM((C,), jnp.int32),
            pltpu.VMEM((C,), jnp.int32),
            pltpu.VMEM((C,), jnp.int32),
            pltpu.VMEM((C, R), jnp.float32),
            pltpu.VMEM((C, R), jnp.float32),
            pltpu.VMEM((C, R), jnp.float32),
            pltpu.VMEM((C, R), jnp.float32),
            pltpu.VMEM((C, R), jnp.float32),
            pltpu.VMEM((C, R), jnp.float32),
            pltpu.VMEM((C, R), jnp.float32),
            pltpu.VMEM((6 * R * L,), jnp.float32),
            pltpu.VMEM((SPW,), jnp.float32),
            pltpu.SemaphoreType.DMA,
        ],
    )
    return f(uidx, sidx, umapA, umapB, smapA, smapB, smapC,
             uemb, uas, ure, semb, sas, sre, spr, prep_flat)


def kernel(userIdx, servIdx, user_as_map, user_re_map, serv_as_map,
           serv_re_map, serv_pr_map, user_emb, uas_emb, ure_emb, serv_emb,
           sas_emb, sre_emb, spr_emb, user_ln_w, user_ln_b, serv_ln_w,
           serv_ln_b, norm_w, norm_b):
    prep = jnp.stack([user_ln_w, user_ln_b, serv_ln_w, serv_ln_b,
                      norm_w, norm_b]).astype(jnp.float32)
    prep_flat = jnp.broadcast_to(prep[:, :, None], (6, R, L)).reshape(-1)
    return _csmf_sc(userIdx, servIdx, user_as_map, user_re_map, serv_as_map,
                    serv_re_map, serv_pr_map, user_emb, uas_emb, ure_emb,
                    serv_emb, sas_emb, sre_emb, spr_emb, prep_flat)


# speedup vs baseline: 1.9444x; 1.1683x over previous
import jax
import jax.numpy as jnp
from jax import lax
from jax.experimental import pallas as pl
from jax.experimental.pallas import tpu as pltpu
from jax.experimental.pallas import tpu_sc as plsc

R = 128
B = 16384
NC = 2
NS = 16
NW = NC * NS
L = 16
SPW = B // NW
C = 32
NCHUNK = SPW // C
NG = C // L
PR = R * L
EPS = 1e-5


def _rsqrt(x):
    i = plsc.bitcast(x, jnp.int32)
    i = jnp.int32(0x5F3759DF) - (i >> 1)
    y = plsc.bitcast(i, jnp.float32)
    for _ in range(3):
        y = y * (1.5 - 0.5 * x * y * y)
    return y


def _sc_body(uidx_hbm, sidx_hbm, umapA_hbm, umapB_hbm,
             smapA_hbm, smapB_hbm, smapC_hbm,
             uemb_hbm, uas_hbm, ure_hbm,
             semb_hbm, sas_hbm, sre_hbm, spr_hbm,
             prep_hbm,
             out_hbm,
             uidx_v, sidx_v, umapA_v, umapB_v, smapA_v, smapB_v, smapC_v,
             uasi_v, urei_v, sasi_v, srei_v, spri_v,
             bu0, buas0, bure0, bs0, bsas0, bsre0, bspr0,
             bu1, buas1, bure1, bs1, bsas1, bsre1, bspr1,
             prep_v, out_v, sem0, sem1, semi):
    wid = lax.axis_index("s") * NC + lax.axis_index("c")
    base = wid * SPW
    bufs = [(bu0, buas0, bure0, bs0, bsas0, bsre0, bspr0),
            (bu1, buas1, bure1, bs1, bsas1, bsre1, bspr1)]
    sems = [sem0, sem1]

    setup = [
        pltpu.async_copy(uidx_hbm.at[pl.ds(base, SPW)], uidx_v, semi),
        pltpu.async_copy(sidx_hbm.at[pl.ds(base, SPW)], sidx_v, semi),
        pltpu.async_copy(umapA_hbm, umapA_v, semi),
        pltpu.async_copy(umapB_hbm, umapB_v, semi),
        pltpu.async_copy(smapA_hbm, smapA_v, semi),
        pltpu.async_copy(smapB_hbm, smapB_v, semi),
        pltpu.async_copy(smapC_hbm, smapC_v, semi),
        pltpu.async_copy(prep_hbm, prep_v, semi),
    ]
    for d in setup:
        d.wait()

    def _sum_param(k):
        def body(f, acc):
            return acc + prep_v[pl.ds(k * PR + f * L, L)]
        return lax.fori_loop(0, R, body, jnp.zeros((L,), jnp.float32))
    Sw = _sum_param(4)
    Sb = _sum_param(5)

    def derive_and_fire(c, s):
        lo = c * C
        for v in range(NG):
            uv = uidx_v[pl.ds(lo + v * L, L)]
            sv = sidx_v[pl.ds(lo + v * L, L)]
            uasi_v[pl.ds(v * L, L)] = plsc.load_gather(umapA_v, [uv])
            urei_v[pl.ds(v * L, L)] = plsc.load_gather(umapB_v, [uv])
            sasi_v[pl.ds(v * L, L)] = plsc.load_gather(smapA_v, [sv])
            srei_v[pl.ds(v * L, L)] = plsc.load_gather(smapB_v, [sv])
            spri_v[pl.ds(v * L, L)] = plsc.load_gather(smapC_v, [sv])
        b = bufs[s]
        sm = sems[s]
        return [
            pltpu.async_copy(uemb_hbm.at[uidx_v.at[pl.ds(lo, C)]], b[0], sm),
            pltpu.async_copy(uas_hbm.at[uasi_v], b[1], sm),
            pltpu.async_copy(ure_hbm.at[urei_v], b[2], sm),
            pltpu.async_copy(semb_hbm.at[sidx_v.at[pl.ds(lo, C)]], b[3], sm),
            pltpu.async_copy(sas_hbm.at[sasi_v], b[4], sm),
            pltpu.async_copy(sre_hbm.at[srei_v], b[5], sm),
            pltpu.async_copy(spr_hbm.at[spri_v], b[6], sm),
        ]

    inv_r = jnp.float32(1.0 / R)
    z = jnp.zeros((L,), jnp.float32)
    descs = [None, None]
    descs[0] = derive_and_fire(0, 0)

    for c in range(NCHUNK):
        s = c % 2
        for d in descs[s]:
            d.wait()
        if c + 1 < NCHUNK:
            descs[1 - s] = derive_and_fire(c + 1, 1 - s)
        b_u, b_uas, b_ure, b_s, b_sas, b_sre, b_spr = bufs[s]

        for g in range(NG):
            rows = lax.iota(jnp.int32, L) + jnp.int32(g * L)

            def load_us(f):
                col = jnp.full((L,), f, jnp.int32)
                u = (plsc.load_gather(b_u, [rows, col])
                     + plsc.load_gather(b_uas, [rows, col])
                     + plsc.load_gather(b_ure, [rows, col]))
                sv = (plsc.load_gather(b_s, [rows, col])
                      + plsc.load_gather(b_sas, [rows, col])
                      + plsc.load_gather(b_sre, [rows, col])
                      + plsc.load_gather(b_spr, [rows, col]))
                return u, sv

            def p1(i, acc):
                su, suu, ss, sss = acc
                for k in range(2):
                    u, sv = load_us(2 * i + k)
                    su = su + u
                    suu = suu + u * u
                    ss = ss + sv
                    sss = sss + sv * sv
                return su, suu, ss, sss
            su, suu, ss, sss = lax.fori_loop(0, R // 2, p1, (z, z, z, z))
            mu = su * inv_r
            ms = ss * inv_r
            iu = _rsqrt(suu * inv_r - mu * mu + EPS)
            isv = _rsqrt(sss * inv_r - ms * ms + EPS)

            def p2(i, acc):
                P, Q, W = acc
                for k in range(2):
                    f = 2 * i + k
                    u, sv = load_us(f)
                    poff = f * L
                    uw = prep_v[pl.ds(poff, L)]
                    ub = prep_v[pl.ds(PR + poff, L)]
                    sw = prep_v[pl.ds(2 * PR + poff, L)]
                    sb = prep_v[pl.ds(3 * PR + poff, L)]
                    w = prep_v[pl.ds(4 * PR + poff, L)]
                    un = (u - mu) * iu * uw + ub
                    sn = (sv - ms) * isv * sw + sb
                    prod = un * sn
                    P = P + prod
                    Q = Q + prod * prod
                    W = W + prod * w
                return P, Q, W
            P, Q, W = lax.fori_loop(0, R // 2, p2, (z, z, z))

            m3 = P * inv_r
            i3 = _rsqrt(Q * inv_r - m3 * m3 + EPS)
            tmp = i3 * (W - m3 * Sw) + Sb
            pred = 1.0 / (1.0 + jnp.exp(-tmp))
            out_v[pl.ds(c * C + g * L, L)] = pred

    pltpu.async_copy(out_v, out_hbm.at[pl.ds(base, SPW)], semi).wait()


@jax.jit
def _csmf_sc(uidx, sidx, umapA, umapB, smapA, smapB, smapC,
             uemb, uas, ure, semb, sas, sre, spr, prep_flat):
    mesh = plsc.VectorSubcoreMesh(core_axis_name="c", subcore_axis_name="s",
                                  num_cores=NC, num_subcores=NS)
    rowbuf = pltpu.VMEM((C, R), jnp.float32)
    idxbuf = pltpu.VMEM((C,), jnp.int32)
    f = pl.kernel(
        _sc_body,
        out_type=jax.ShapeDtypeStruct((B,), jnp.float32),
        mesh=mesh,
        compiler_params=pltpu.CompilerParams(needs_layout_passes=False),
        scratch_types=(
            [pltpu.VMEM((SPW,), jnp.int32)] * 2
            + [pltpu.VMEM((339,), jnp.int32)] * 2
            + [pltpu.VMEM((5825,), jnp.int32)] * 3
            + [idxbuf] * 5
            + [rowbuf] * 14
            + [pltpu.VMEM((6 * PR,), jnp.float32),
               pltpu.VMEM((SPW,), jnp.float32)]
            + [pltpu.SemaphoreType.DMA] * 3
        ),
    )
    return f(uidx, sidx, umapA, umapB, smapA, smapB, smapC,
             uemb, uas, ure, semb, sas, sre, spr, prep_flat)


def kernel(userIdx, servIdx, user_as_map, user_re_map, serv_as_map,
           serv_re_map, serv_pr_map, user_emb, uas_emb, ure_emb, serv_emb,
           sas_emb, sre_emb, spr_emb, user_ln_w, user_ln_b, serv_ln_w,
           serv_ln_b, norm_w, norm_b):
    prep = jnp.stack([user_ln_w, user_ln_b, serv_ln_w, serv_ln_b,
                      norm_w, norm_b]).astype(jnp.float32)
    prep_flat = jnp.broadcast_to(prep[:, :, None], (6, R, L)).reshape(-1)
    return _csmf_sc(userIdx, servIdx, user_as_map, user_re_map, serv_as_map,
                    serv_re_map, serv_pr_map, user_emb, uas_emb, ure_emb,
                    serv_emb, sas_emb, sre_emb, spr_emb, prep_flat)


# speedup vs baseline: 11.0923x; 5.7048x over previous
import jax
import jax.numpy as jnp
from jax import lax
from jax.experimental import pallas as pl
from jax.experimental.pallas import tpu as pltpu
from jax.experimental.pallas import tpu_sc as plsc

R = 128
B = 16384
NC = 2
NS = 16
NW = NC * NS
L = 16
SPW = B // NW
C = 32
NCHUNK = SPW // C
NG = C // L
EPS = 1e-5


def _rsqrt(x):
    i = plsc.bitcast(x, jnp.int32)
    i = jnp.int32(0x5F3759DF) - (i >> 1)
    y = plsc.bitcast(i, jnp.float32)
    for _ in range(3):
        y = y * (1.5 - 0.5 * x * y * y)
    return y


def _sc_body(uidx_hbm, sidx_hbm, umapA_hbm, umapB_hbm,
             smapA_hbm, smapB_hbm, smapC_hbm,
             uemb_hbm, uas_hbm, ure_hbm,
             semb_hbm, sas_hbm, sre_hbm, spr_hbm,
             prm_hbm,
             out_hbm,
             uidx_v, sidx_v, umapA_v, umapB_v, smapA_v, smapB_v, smapC_v,
             uasi_v, urei_v, sasi_v, srei_v, spri_v,
             bu0, buas0, bure0, bs0, bsas0, bsre0, bspr0,
             bu1, buas1, bure1, bs1, bsas1, bsre1, bspr1,
             prm_v, out_v, sem0, sem1, semi):
    wid = lax.axis_index("s") * NC + lax.axis_index("c")
    base = wid * SPW
    bufs = [(bu0, buas0, bure0, bs0, bsas0, bsre0, bspr0),
            (bu1, buas1, bure1, bs1, bsas1, bsre1, bspr1)]
    sems = [sem0, sem1]

    setup = [
        pltpu.async_copy(uidx_hbm.at[pl.ds(base, SPW)], uidx_v, semi),
        pltpu.async_copy(sidx_hbm.at[pl.ds(base, SPW)], sidx_v, semi),
        pltpu.async_copy(umapA_hbm, umapA_v, semi),
        pltpu.async_copy(umapB_hbm, umapB_v, semi),
        pltpu.async_copy(smapA_hbm, smapA_v, semi),
        pltpu.async_copy(smapB_hbm, smapB_v, semi),
        pltpu.async_copy(smapC_hbm, smapC_v, semi),
        pltpu.async_copy(prm_hbm, prm_v, semi),
    ]
    for d in setup:
        d.wait()

    def _sum_param(k):
        acc = jnp.zeros((L,), jnp.float32)
        for j in range(R // L):
            acc = acc + prm_v[k, pl.ds(j * L, L)]
        return jnp.sum(acc)
    Sw = _sum_param(4)
    Sb = _sum_param(5)

    def derive_and_fire(c, s):
        lo = c * C
        for v in range(NG):
            uv = uidx_v[pl.ds(lo + v * L, L)]
            sv = sidx_v[pl.ds(lo + v * L, L)]
            uasi_v[pl.ds(v * L, L)] = plsc.load_gather(umapA_v, [uv])
            urei_v[pl.ds(v * L, L)] = plsc.load_gather(umapB_v, [uv])
            sasi_v[pl.ds(v * L, L)] = plsc.load_gather(smapA_v, [sv])
            srei_v[pl.ds(v * L, L)] = plsc.load_gather(smapB_v, [sv])
            spri_v[pl.ds(v * L, L)] = plsc.load_gather(smapC_v, [sv])
        b = bufs[s]
        sm = sems[s]
        return [
            pltpu.async_copy(uemb_hbm.at[uidx_v.at[pl.ds(lo, C)]], b[0], sm),
            pltpu.async_copy(uas_hbm.at[uasi_v], b[1], sm),
            pltpu.async_copy(ure_hbm.at[urei_v], b[2], sm),
            pltpu.async_copy(semb_hbm.at[sidx_v.at[pl.ds(lo, C)]], b[3], sm),
            pltpu.async_copy(sas_hbm.at[sasi_v], b[4], sm),
            pltpu.async_copy(sre_hbm.at[srei_v], b[5], sm),
            pltpu.async_copy(spr_hbm.at[spri_v], b[6], sm),
        ]

    inv_r = jnp.float32(1.0 / R)
    z = jnp.zeros((L,), jnp.float32)
    lanes = lax.iota(jnp.int32, L)
    rows = [lanes + jnp.int32(g * L) for g in range(NG)]
    k_idx = [jnp.full((L,), k, jnp.int32) for k in range(5)]
    descs = [None, None]
    descs[0] = derive_and_fire(0, 0)

    for c in range(NCHUNK):
        s = c % 2
        for d in descs[s]:
            d.wait()
        if c + 1 < NCHUNK:
            descs[1 - s] = derive_and_fire(c + 1, 1 - s)
        b = bufs[s]

        def load_us(g, col):
            u = (plsc.load_gather(b[0], [rows[g], col])
                 + plsc.load_gather(b[1], [rows[g], col])
                 + plsc.load_gather(b[2], [rows[g], col]))
            sv = (plsc.load_gather(b[3], [rows[g], col])
                  + plsc.load_gather(b[4], [rows[g], col])
                  + plsc.load_gather(b[5], [rows[g], col])
                  + plsc.load_gather(b[6], [rows[g], col]))
            return u, sv

        def p1b(i, acc):
            moms, col = acc[:-1], acc[-1]
            moms = list(moms)
            for k in range(2):
                for g in range(NG):
                    su, suu, ss, sss = moms[g]
                    u, sv = load_us(g, col)
                    moms[g] = (su + u, suu + u * u, ss + sv, sss + sv * sv)
                col = (col + 1) & jnp.int32(127)
            return tuple(moms) + (col,)

        init = tuple(((z, z, z, z)) for _ in range(NG)) + (lanes,)
        res = lax.fori_loop(0, R // 2, p1b, init)
        stats = []
        for g in range(NG):
            su, suu, ss, sss = res[g]
            mu = su * inv_r
            ms = ss * inv_r
            iu = _rsqrt(suu * inv_r - mu * mu + EPS)
            isv = _rsqrt(sss * inv_r - ms * ms + EPS)
            stats.append((mu, ms, iu, isv))

        def p2(i, acc):
            moms, col = acc[:-1], acc[-1]
            moms = list(moms)
            for k in range(2):
                uw = plsc.load_gather(prm_v, [k_idx[0], col])
                ub = plsc.load_gather(prm_v, [k_idx[1], col])
                sw = plsc.load_gather(prm_v, [k_idx[2], col])
                sb = plsc.load_gather(prm_v, [k_idx[3], col])
                w = plsc.load_gather(prm_v, [k_idx[4], col])
                for g in range(NG):
                    mu, ms, iu, isv = stats[g]
                    P, Q, W = moms[g]
                    u, sv = load_us(g, col)
                    un = (u - mu) * iu * uw + ub
                    sn = (sv - ms) * isv * sw + sb
                    prod = un * sn
                    moms[g] = (P + prod, Q + prod * prod, W + prod * w)
                col = (col + 1) & jnp.int32(127)
            return tuple(moms) + (col,)

        init2 = tuple(((z, z, z)) for _ in range(NG)) + (lanes,)
        res2 = lax.fori_loop(0, R // 2, p2, init2)
        for g in range(NG):
            P, Q, W = res2[g]
            m3 = P * inv_r
            i3 = _rsqrt(Q * inv_r - m3 * m3 + EPS)
            tmp = i3 * (W - m3 * Sw) + Sb
            pred = 1.0 / (1.0 + jnp.exp(-tmp))
            out_v[pl.ds(c * C + g * L, L)] = pred

    pltpu.async_copy(out_v, out_hbm.at[pl.ds(base, SPW)], semi).wait()


@jax.jit
def _csmf_sc(uidx, sidx, umapA, umapB, smapA, smapB, smapC,
             uemb, uas, ure, semb, sas, sre, spr, prm):
    mesh = plsc.VectorSubcoreMesh(core_axis_name="c", subcore_axis_name="s",
                                  num_cores=NC, num_subcores=NS)
    rowbuf = pltpu.VMEM((C, R), jnp.float32)
    idxbuf = pltpu.VMEM((C,), jnp.int32)
    f = pl.kernel(
        _sc_body,
        out_type=jax.ShapeDtypeStruct((B,), jnp.float32),
        mesh=mesh,
        compiler_params=pltpu.CompilerParams(needs_layout_passes=False),
        scratch_types=(
            [pltpu.VMEM((SPW,), jnp.int32)] * 2
            + [pltpu.VMEM((339,), jnp.int32)] * 2
            + [pltpu.VMEM((5825,), jnp.int32)] * 3
            + [idxbuf] * 5
            + [rowbuf] * 14
            + [pltpu.VMEM((6, R), jnp.float32),
               pltpu.VMEM((SPW,), jnp.float32)]
            + [pltpu.SemaphoreType.DMA] * 3
        ),
    )
    return f(uidx, sidx, umapA, umapB, smapA, smapB, smapC,
             uemb, uas, ure, semb, sas, sre, spr, prm)


def kernel(userIdx, servIdx, user_as_map, user_re_map, serv_as_map,
           serv_re_map, serv_pr_map, user_emb, uas_emb, ure_emb, serv_emb,
           sas_emb, sre_emb, spr_emb, user_ln_w, user_ln_b, serv_ln_w,
           serv_ln_b, norm_w, norm_b):
    prm = jnp.stack([user_ln_w, user_ln_b, serv_ln_w, serv_ln_b,
                     norm_w, norm_b]).astype(jnp.float32)
    return _csmf_sc(userIdx, servIdx, user_as_map, user_re_map, serv_as_map,
                    serv_re_map, serv_pr_map, user_emb, uas_emb, ure_emb,
                    serv_emb, sas_emb, sre_emb, spr_emb, prm)
